# baseline (speedup 1.0000x reference)
import jax
import jax.numpy as jnp
from jax import lax
from jax.experimental import pallas as pl
from jax.experimental.pallas import tpu as pltpu

GROUPS = 4
NUM_Q = 4
K = 1024
DG = 64
TOKENS = 8192
TBLK = 512


def _sumsq64(v):
    v = v * v
    acc = v[:, 0:8]
    for j in range(1, 8):
        acc = acc + v[:, 8 * j:8 * j + 8]
    t = acc[:, :4] + acc[:, 4:]
    t = t[:, :2] + t[:, 2:]
    return t[:, :1] + t[:, 1:2]


def _vq_kernel(x_ref, cb_ref, out_ref, idx_ref, closs_ref, cn_ref):
    i = pl.program_id(0)

    @pl.when(i == 0)
    def _():
        for g in range(GROUPS):
            for q in range(NUM_Q):
                col = _sumsq64(cb_ref[g, q])
                cn_ref[pl.ds(g * NUM_Q + q, 1), :] = col.T

    xb = x_ref[...]
    ids = lax.broadcasted_iota(jnp.int32, (TBLK, K), 1)

    group_out = []
    closs_cols = []
    for g in range(GROUPS):
        residual = xb[:, g * DG:(g + 1) * DG]
        qout = jnp.zeros_like(residual)
        for q in range(NUM_Q):
            l = g * NUM_Q + q
            cb = cb_ref[g, q]
            scores = lax.dot_general(
                residual, cb, (((1,), (1,)), ((), ())),
                preferred_element_type=jnp.float32)
            rnorm = _sumsq64(residual)
            d2 = (rnorm - 2.0 * scores) + cn_ref[pl.ds(l, 1), :]
            m = jnp.min(d2, axis=1, keepdims=True)
            idx = jnp.min(jnp.where(d2 <= m, ids, K), axis=1, keepdims=True)
            idx_ref[:, pl.ds(l, 1)] = idx
            onehot = (ids == idx).astype(jnp.float32)
            quant = lax.dot_general(
                onehot, cb, (((1,), (0,)), ((), ())),
                precision=lax.Precision.HIGHEST,
                preferred_element_type=jnp.float32)
            new_residual = residual - quant
            closs_cols.append(jnp.sum(new_residual * new_residual))
            qout = qout + quant
            residual = new_residual
        group_out.append(qout)

    out_ref[...] = jnp.concatenate(group_out, axis=1)
    closs_row = jnp.concatenate(
        [jnp.full((1, 1), c, jnp.float32) for c in closs_cols], axis=1)

    @pl.when(i == 0)
    def _():
        closs_ref[...] = closs_row

    @pl.when(i > 0)
    def _():
        closs_ref[...] = closs_ref[...] + closs_row


@jax.jit
def kernel(x, codebooks):
    B, N, D = x.shape
    x2 = x.reshape(TOKENS, D)
    grid = TOKENS // TBLK
    out, idx, closs = pl.pallas_call(
        _vq_kernel,
        grid=(grid,),
        in_specs=[
            pl.BlockSpec((TBLK, D), lambda i: (i, 0)),
            pl.BlockSpec((GROUPS, NUM_Q, K, DG), lambda i: (0, 0, 0, 0)),
        ],
        out_specs=[
            pl.BlockSpec((TBLK, D), lambda i: (i, 0)),
            pl.BlockSpec((TBLK, GROUPS * NUM_Q), lambda i: (i, 0)),
            pl.BlockSpec((1, GROUPS * NUM_Q), lambda i: (0, 0)),
        ],
        out_shape=[
            jax.ShapeDtypeStruct((TOKENS, D), jnp.float32),
            jax.ShapeDtypeStruct((TOKENS, GROUPS * NUM_Q), jnp.int32),
            jax.ShapeDtypeStruct((1, GROUPS * NUM_Q), jnp.float32),
        ],
        scratch_shapes=[pltpu.VMEM((GROUPS * NUM_Q, K), jnp.float32)],
    )(x2, codebooks)

    quantized = out.reshape(B, N, D)
    all_indices = idx.reshape(B, N, GROUPS, NUM_Q).transpose(2, 0, 1, 3)
    commit_losses = closs.reshape(GROUPS, NUM_Q) / (TOKENS * DG)
    return quantized, all_indices, commit_losses


# speedup vs baseline: 1.3056x; 1.3056x over previous
import jax
import jax.numpy as jnp
from jax import lax
from jax.experimental import pallas as pl
from jax.experimental.pallas import tpu as pltpu

GROUPS = 4
NUM_Q = 4
K = 1024
DG = 64
TOKENS = 8192
TBLK = 512


def _sumsq64(v):
    v = v * v
    acc = v[:, 0:8]
    for j in range(1, 8):
        acc = acc + v[:, 8 * j:8 * j + 8]
    t = acc[:, :4] + acc[:, 4:]
    t = t[:, :2] + t[:, 2:]
    return t[:, :1] + t[:, 1:2]


def _vq_kernel(x_ref, cb_ref, hi_ref, mid_ref, lo_ref,
               out_ref, idx_ref, closs_ref, cn_ref):
    i = pl.program_id(0)

    @pl.when(i == 0)
    def _():
        for g in range(GROUPS):
            for q in range(NUM_Q):
                col = _sumsq64(cb_ref[g, q])
                cn_ref[pl.ds(g * NUM_Q + q, 1), :] = col.T

    xb = x_ref[...]
    ids = lax.broadcasted_iota(jnp.int32, (TBLK, K), 1)

    group_out = []
    closs_cols = []
    for g in range(GROUPS):
        residual = xb[:, g * DG:(g + 1) * DG]
        qout = jnp.zeros_like(residual)
        for q in range(NUM_Q):
            l = g * NUM_Q + q
            cb = cb_ref[g, q]
            scores = lax.dot_general(
                residual, cb, (((1,), (1,)), ((), ())),
                preferred_element_type=jnp.float32)
            rnorm = _sumsq64(residual)
            d2 = (rnorm - 2.0 * scores) + cn_ref[pl.ds(l, 1), :]
            m = jnp.min(d2, axis=1, keepdims=True)
            idx = jnp.min(jnp.where(d2 <= m, ids, K), axis=1, keepdims=True)
            idx_ref[:, pl.ds(l, 1)] = idx
            onehot = (ids == idx).astype(jnp.bfloat16)
            quant = (lax.dot_general(
                onehot, hi_ref[g, q], (((1,), (0,)), ((), ())),
                preferred_element_type=jnp.float32)
                + lax.dot_general(
                onehot, mid_ref[g, q], (((1,), (0,)), ((), ())),
                preferred_element_type=jnp.float32)) \
                + lax.dot_general(
                onehot, lo_ref[g, q], (((1,), (0,)), ((), ())),
                preferred_element_type=jnp.float32)
            new_residual = residual - quant
            closs_cols.append(jnp.sum(new_residual * new_residual))
            qout = qout + quant
            residual = new_residual
        group_out.append(qout)

    out_ref[...] = jnp.concatenate(group_out, axis=1)
    closs_row = jnp.concatenate(
        [jnp.full((1, 1), c, jnp.float32) for c in closs_cols], axis=1)

    @pl.when(i == 0)
    def _():
        closs_ref[...] = closs_row

    @pl.when(i > 0)
    def _():
        closs_ref[...] = closs_ref[...] + closs_row


def _trunc_bf16(v):
    u = lax.bitcast_convert_type(v, jnp.uint32)
    return lax.bitcast_convert_type(u & jnp.uint32(0xFFFF0000), jnp.float32)


@jax.jit
def kernel(x, codebooks):
    B, N, D = x.shape
    x2 = x.reshape(TOKENS, D)
    hi_f = _trunc_bf16(codebooks)
    r1 = codebooks - hi_f
    mid_f = _trunc_bf16(r1)
    lo_f = r1 - mid_f
    cb_hi = hi_f.astype(jnp.bfloat16)
    cb_mid = mid_f.astype(jnp.bfloat16)
    cb_lo = lo_f.astype(jnp.bfloat16)
    grid = TOKENS // TBLK
    cb_spec = pl.BlockSpec((GROUPS, NUM_Q, K, DG), lambda i: (0, 0, 0, 0))
    out, idx, closs = pl.pallas_call(
        _vq_kernel,
        grid=(grid,),
        in_specs=[
            pl.BlockSpec((TBLK, D), lambda i: (i, 0)),
            cb_spec, cb_spec, cb_spec, cb_spec,
        ],
        out_specs=[
            pl.BlockSpec((TBLK, D), lambda i: (i, 0)),
            pl.BlockSpec((TBLK, GROUPS * NUM_Q), lambda i: (i, 0)),
            pl.BlockSpec((1, GROUPS * NUM_Q), lambda i: (0, 0)),
        ],
        out_shape=[
            jax.ShapeDtypeStruct((TOKENS, D), jnp.float32),
            jax.ShapeDtypeStruct((TOKENS, GROUPS * NUM_Q), jnp.int32),
            jax.ShapeDtypeStruct((1, GROUPS * NUM_Q), jnp.float32),
        ],
        scratch_shapes=[pltpu.VMEM((GROUPS * NUM_Q, K), jnp.float32)],
    )(x2, codebooks, cb_hi, cb_mid, cb_lo)

    quantized = out.reshape(B, N, D)
    all_indices = idx.reshape(B, N, GROUPS, NUM_Q).transpose(2, 0, 1, 3)
    commit_losses = closs.reshape(GROUPS, NUM_Q) / (TOKENS * DG)
    return quantized, all_indices, commit_losses


# speedup vs baseline: 1.3596x; 1.0414x over previous
import jax
import jax.numpy as jnp
from jax import lax
from jax.experimental import pallas as pl
from jax.experimental.pallas import tpu as pltpu

GROUPS = 4
NUM_Q = 4
K = 1024
DG = 64
TOKENS = 8192
TBLK = 512


def _sumsq64(v):
    v = v * v
    acc = v[:, 0:8]
    for j in range(1, 8):
        acc = acc + v[:, 8 * j:8 * j + 8]
    t = acc[:, :4] + acc[:, 4:]
    t = t[:, :2] + t[:, 2:]
    return t[:, :1] + t[:, 1:2]


def _vq_kernel(x_ref, cbs_ref, hi_ref, mid_ref, lo_ref,
               out_ref, idx_ref, closs_ref, cn_ref):
    i = pl.program_id(0)

    @pl.when(i == 0)
    def _():
        for g in range(GROUPS):
            for q in range(NUM_Q):
                cbf = (hi_ref[g, q].astype(jnp.float32)
                       + mid_ref[g, q].astype(jnp.float32)) \
                    + lo_ref[g, q].astype(jnp.float32)
                col = _sumsq64(cbf)
                cn_ref[pl.ds(g * NUM_Q + q, 1), :] = col.T

    xb = x_ref[...]
    ids = lax.broadcasted_iota(jnp.int32, (TBLK, K), 1)

    residual = [xb[:, g * DG:(g + 1) * DG] for g in range(GROUPS)]
    qout = [jnp.zeros((TBLK, DG), jnp.float32) for _ in range(GROUPS)]
    closs_cols = [None] * (GROUPS * NUM_Q)
    for q in range(NUM_Q):
        for g in range(GROUPS):
            l = g * NUM_Q + q
            r = residual[g]
            rb2 = (r + r).astype(jnp.bfloat16)
            scores2 = lax.dot_general(
                rb2, cbs_ref[g, q], (((1,), (1,)), ((), ())),
                preferred_element_type=jnp.float32)
            rnorm = _sumsq64(r)
            d2 = (rnorm - scores2) + cn_ref[pl.ds(l, 1), :]
            m = jnp.min(d2, axis=1, keepdims=True)
            idx = jnp.min(jnp.where(d2 <= m, ids, K), axis=1, keepdims=True)
            idx_ref[:, pl.ds(l, 1)] = idx
            onehot = (ids == idx).astype(jnp.bfloat16)
            quant = (lax.dot_general(
                onehot, hi_ref[g, q], (((1,), (0,)), ((), ())),
                preferred_element_type=jnp.float32)
                + lax.dot_general(
                onehot, mid_ref[g, q], (((1,), (0,)), ((), ())),
                preferred_element_type=jnp.float32)) \
                + lax.dot_general(
                onehot, lo_ref[g, q], (((1,), (0,)), ((), ())),
                preferred_element_type=jnp.float32)
            new_r = r - quant
            closs_cols[l] = jnp.sum(new_r * new_r)
            qout[g] = qout[g] + quant
            residual[g] = new_r

    out_ref[...] = jnp.concatenate(qout, axis=1)
    closs_row = jnp.concatenate(
        [jnp.full((1, 1), c, jnp.float32) for c in closs_cols], axis=1)

    @pl.when(i == 0)
    def _():
        closs_ref[...] = closs_row

    @pl.when(i > 0)
    def _():
        closs_ref[...] = closs_ref[...] + closs_row


def _trunc_bf16(v):
    u = lax.bitcast_convert_type(v, jnp.uint32)
    return lax.bitcast_convert_type(u & jnp.uint32(0xFFFF0000), jnp.float32)


@jax.jit
def kernel(x, codebooks):
    B, N, D = x.shape
    x2 = x.reshape(TOKENS, D)
    cb_s = codebooks.astype(jnp.bfloat16)
    hi_f = _trunc_bf16(codebooks)
    r1 = codebooks - hi_f
    mid_f = _trunc_bf16(r1)
    lo_f = r1 - mid_f
    cb_hi = hi_f.astype(jnp.bfloat16)
    cb_mid = mid_f.astype(jnp.bfloat16)
    cb_lo = lo_f.astype(jnp.bfloat16)
    grid = TOKENS // TBLK
    cb_spec = pl.BlockSpec((GROUPS, NUM_Q, K, DG), lambda i: (0, 0, 0, 0))
    out, idx, closs = pl.pallas_call(
        _vq_kernel,
        grid=(grid,),
        in_specs=[
            pl.BlockSpec((TBLK, D), lambda i: (i, 0)),
            cb_spec, cb_spec, cb_spec, cb_spec,
        ],
        out_specs=[
            pl.BlockSpec((TBLK, D), lambda i: (i, 0)),
            pl.BlockSpec((TBLK, GROUPS * NUM_Q), lambda i: (i, 0)),
            pl.BlockSpec((1, GROUPS * NUM_Q), lambda i: (0, 0)),
        ],
        out_shape=[
            jax.ShapeDtypeStruct((TOKENS, D), jnp.float32),
            jax.ShapeDtypeStruct((TOKENS, GROUPS * NUM_Q), jnp.int32),
            jax.ShapeDtypeStruct((1, GROUPS * NUM_Q), jnp.float32),
        ],
        scratch_shapes=[pltpu.VMEM((GROUPS * NUM_Q, K), jnp.float32)],
    )(x2, cb_s, cb_hi, cb_mid, cb_lo)

    quantized = out.reshape(B, N, D)
    all_indices = idx.reshape(B, N, GROUPS, NUM_Q).transpose(2, 0, 1, 3)
    commit_losses = closs.reshape(GROUPS, NUM_Q) / (TOKENS * DG)
    return quantized, all_indices, commit_losses


# speedup vs baseline: 1.3744x; 1.0109x over previous
import jax
import jax.numpy as jnp
from jax import lax
from jax.experimental import pallas as pl
from jax.experimental.pallas import tpu as pltpu

GROUPS = 4
NUM_Q = 4
K = 1024
DG = 64
TOKENS = 8192
TBLK = 512


def _sumsq64(v):
    v = v * v
    acc = v[:, 0:8]
    for j in range(1, 8):
        acc = acc + v[:, 8 * j:8 * j + 8]
    t = acc[:, :4] + acc[:, 4:]
    t = t[:, :2] + t[:, 2:]
    return t[:, :1] + t[:, 1:2]


def _vq_kernel(x_ref, cbs_ref, hi_ref, mid_ref, lo_ref,
               out_ref, idx_ref, closs_ref, cn_ref):
    i = pl.program_id(0)

    @pl.when(i == 0)
    def _():
        for g in range(GROUPS):
            for q in range(NUM_Q):
                cbf = (hi_ref[g, q].astype(jnp.float32)
                       + mid_ref[g, q].astype(jnp.float32)) \
                    + lo_ref[g, q].astype(jnp.float32)
                col = _sumsq64(cbf)
                cn_ref[pl.ds(g * NUM_Q + q, 1), :] = col.T

    xb = x_ref[...]
    ids = lax.broadcasted_iota(jnp.int32, (TBLK, K), 1).astype(jnp.float32)

    residual = [xb[:, g * DG:(g + 1) * DG] for g in range(GROUPS)]
    qout = [jnp.zeros((TBLK, DG), jnp.float32) for _ in range(GROUPS)]
    closs_cols = [None] * (GROUPS * NUM_Q)
    for q in range(NUM_Q):
        for g in range(GROUPS):
            l = g * NUM_Q + q
            r = residual[g]
            rb2 = (r + r).astype(jnp.bfloat16)
            scores2 = lax.dot_general(
                rb2, cbs_ref[g, q], (((1,), (1,)), ((), ())),
                preferred_element_type=jnp.float32)
            rnorm = _sumsq64(r)
            d2 = (rnorm - scores2) + cn_ref[pl.ds(l, 1), :]
            m = jnp.min(d2, axis=1, keepdims=True)
            idxf = jnp.min(jnp.where(d2 <= m, ids, float(K)), axis=1,
                           keepdims=True)
            idx_ref[:, pl.ds(l, 1)] = idxf.astype(jnp.int32)
            onehot = (ids == idxf).astype(jnp.bfloat16)
            quant = (lax.dot_general(
                onehot, hi_ref[g, q], (((1,), (0,)), ((), ())),
                preferred_element_type=jnp.float32)
                + lax.dot_general(
                onehot, mid_ref[g, q], (((1,), (0,)), ((), ())),
                preferred_element_type=jnp.float32)) \
                + lax.dot_general(
                onehot, lo_ref[g, q], (((1,), (0,)), ((), ())),
                preferred_element_type=jnp.float32)
            new_r = r - quant
            closs_cols[l] = jnp.sum(new_r * new_r)
            qout[g] = qout[g] + quant
            residual[g] = new_r

    out_ref[...] = jnp.concatenate(qout, axis=1)
    closs_row = jnp.concatenate(
        [jnp.full((1, 1), c, jnp.float32) for c in closs_cols], axis=1)

    @pl.when(i == 0)
    def _():
        closs_ref[...] = closs_row

    @pl.when(i > 0)
    def _():
        closs_ref[...] = closs_ref[...] + closs_row


def _trunc_bf16(v):
    u = lax.bitcast_convert_type(v, jnp.uint32)
    return lax.bitcast_convert_type(u & jnp.uint32(0xFFFF0000), jnp.float32)


@jax.jit
def kernel(x, codebooks):
    B, N, D = x.shape
    x2 = x.reshape(TOKENS, D)
    cb_s = codebooks.astype(jnp.bfloat16)
    hi_f = _trunc_bf16(codebooks)
    r1 = codebooks - hi_f
    mid_f = _trunc_bf16(r1)
    lo_f = r1 - mid_f
    cb_hi = hi_f.astype(jnp.bfloat16)
    cb_mid = mid_f.astype(jnp.bfloat16)
    cb_lo = lo_f.astype(jnp.bfloat16)
    grid = TOKENS // TBLK
    cb_spec = pl.BlockSpec((GROUPS, NUM_Q, K, DG), lambda i: (0, 0, 0, 0))
    out, idx, closs = pl.pallas_call(
        _vq_kernel,
        grid=(grid,),
        in_specs=[
            pl.BlockSpec((TBLK, D), lambda i: (i, 0)),
            cb_spec, cb_spec, cb_spec, cb_spec,
        ],
        out_specs=[
            pl.BlockSpec((TBLK, D), lambda i: (i, 0)),
            pl.BlockSpec((TBLK, GROUPS * NUM_Q), lambda i: (i, 0)),
            pl.BlockSpec((1, GROUPS * NUM_Q), lambda i: (0, 0)),
        ],
        out_shape=[
            jax.ShapeDtypeStruct((TOKENS, D), jnp.float32),
            jax.ShapeDtypeStruct((TOKENS, GROUPS * NUM_Q), jnp.int32),
            jax.ShapeDtypeStruct((1, GROUPS * NUM_Q), jnp.float32),
        ],
        scratch_shapes=[pltpu.VMEM((GROUPS * NUM_Q, K), jnp.float32)],
    )(x2, cb_s, cb_hi, cb_mid, cb_lo)

    quantized = out.reshape(B, N, D)
    all_indices = idx.reshape(B, N, GROUPS, NUM_Q).transpose(2, 0, 1, 3)
    commit_losses = closs.reshape(GROUPS, NUM_Q) / (TOKENS * DG)
    return quantized, all_indices, commit_losses


# speedup vs baseline: 1.5548x; 1.1313x over previous
import jax
import jax.numpy as jnp
from jax import lax
from jax.experimental import pallas as pl
from jax.experimental.pallas import tpu as pltpu

GROUPS = 4
NUM_Q = 4
K = 1024
DG = 64
TOKENS = 8192
TBLK = 512


def _sumsq64(v):
    v = v * v
    acc = v[:, 0:8]
    for j in range(1, 8):
        acc = acc + v[:, 8 * j:8 * j + 8]
    t = acc[:, :4] + acc[:, 4:]
    t = t[:, :2] + t[:, 2:]
    return t[:, :1] + t[:, 1:2]


def _vq_kernel(x_ref, cbs_ref, gat_ref,
               out_ref, idx_ref, closs_ref, cn_ref):
    i = pl.program_id(0)

    @pl.when(i == 0)
    def _():
        for g in range(GROUPS):
            for q in range(NUM_Q):
                gat = gat_ref[g, q].astype(jnp.float32)
                cbf = (gat[:, :DG] + gat[:, DG:2 * DG]) + gat[:, 2 * DG:]
                col = _sumsq64(cbf)
                cn_ref[pl.ds(g * NUM_Q + q, 1), :] = col.T

    xb = x_ref[...]
    ids = lax.broadcasted_iota(jnp.int32, (TBLK, K), 1).astype(jnp.float32)

    residual = [xb[:, g * DG:(g + 1) * DG] for g in range(GROUPS)]
    qout = [jnp.zeros((TBLK, DG), jnp.float32) for _ in range(GROUPS)]
    closs_cols = [None] * (GROUPS * NUM_Q)
    for q in range(NUM_Q):
        for g in range(GROUPS):
            l = g * NUM_Q + q
            r = residual[g]
            rb2 = (r + r).astype(jnp.bfloat16)
            scores2 = lax.dot_general(
                rb2, cbs_ref[g, q], (((1,), (1,)), ((), ())),
                preferred_element_type=jnp.float32)
            rnorm = _sumsq64(r)
            d2 = (rnorm - scores2) + cn_ref[pl.ds(l, 1), :]
            m = jnp.min(d2, axis=1, keepdims=True)
            idxf = jnp.min(jnp.where(d2 <= m, ids, float(K)), axis=1,
                           keepdims=True)
            idx_ref[:, pl.ds(l, 1)] = idxf.astype(jnp.int32)
            onehot = (ids == idxf).astype(jnp.bfloat16)
            gq = lax.dot_general(
                onehot, gat_ref[g, q], (((1,), (0,)), ((), ())),
                preferred_element_type=jnp.float32)
            quant = (gq[:, :DG] + gq[:, DG:2 * DG]) + gq[:, 2 * DG:]
            new_r = r - quant
            closs_cols[l] = jnp.sum(new_r * new_r)
            qout[g] = qout[g] + quant
            residual[g] = new_r

    out_ref[...] = jnp.concatenate(qout, axis=1)
    closs_row = jnp.concatenate(
        [jnp.full((1, 1), c, jnp.float32) for c in closs_cols], axis=1)

    @pl.when(i == 0)
    def _():
        closs_ref[...] = closs_row

    @pl.when(i > 0)
    def _():
        closs_ref[...] = closs_ref[...] + closs_row


def _trunc_bf16(v):
    u = lax.bitcast_convert_type(v, jnp.uint32)
    return lax.bitcast_convert_type(u & jnp.uint32(0xFFFF0000), jnp.float32)


@jax.jit
def kernel(x, codebooks):
    B, N, D = x.shape
    x2 = x.reshape(TOKENS, D)
    cb_s = codebooks.astype(jnp.bfloat16)
    hi_f = _trunc_bf16(codebooks)
    r1 = codebooks - hi_f
    mid_f = _trunc_bf16(r1)
    lo_f = r1 - mid_f
    cb_gat = jnp.concatenate(
        [hi_f.astype(jnp.bfloat16), mid_f.astype(jnp.bfloat16),
         lo_f.astype(jnp.bfloat16)], axis=-1)
    grid = TOKENS // TBLK
    out, idx, closs = pl.pallas_call(
        _vq_kernel,
        grid=(grid,),
        in_specs=[
            pl.BlockSpec((TBLK, D), lambda i: (i, 0)),
            pl.BlockSpec((GROUPS, NUM_Q, K, DG), lambda i: (0, 0, 0, 0)),
            pl.BlockSpec((GROUPS, NUM_Q, K, 3 * DG), lambda i: (0, 0, 0, 0)),
        ],
        out_specs=[
            pl.BlockSpec((TBLK, D), lambda i: (i, 0)),
            pl.BlockSpec((TBLK, GROUPS * NUM_Q), lambda i: (i, 0)),
            pl.BlockSpec((1, GROUPS * NUM_Q), lambda i: (0, 0)),
        ],
        out_shape=[
            jax.ShapeDtypeStruct((TOKENS, D), jnp.float32),
            jax.ShapeDtypeStruct((TOKENS, GROUPS * NUM_Q), jnp.int32),
            jax.ShapeDtypeStruct((1, GROUPS * NUM_Q), jnp.float32),
        ],
        scratch_shapes=[pltpu.VMEM((GROUPS * NUM_Q, K), jnp.float32)],
    )(x2, cb_s, cb_gat)

    quantized = out.reshape(B, N, D)
    all_indices = idx.reshape(B, N, GROUPS, NUM_Q).transpose(2, 0, 1, 3)
    commit_losses = closs.reshape(GROUPS, NUM_Q) / (TOKENS * DG)
    return quantized, all_indices, commit_losses


# speedup vs baseline: 2.0566x; 1.3227x over previous
import jax
import jax.numpy as jnp
from jax import lax
from jax.experimental import pallas as pl
from jax.experimental.pallas import tpu as pltpu

GROUPS = 4
NUM_Q = 4
K = 1024
DG = 64
TOKENS = 8192
TBLK = 256


def _sumsq64(v):
    v = v * v
    acc = v[:, 0:8]
    for j in range(1, 8):
        acc = acc + v[:, 8 * j:8 * j + 8]
    t = acc[:, :4] + acc[:, 4:]
    t = t[:, :2] + t[:, 2:]
    return t[:, :1] + t[:, 1:2]


def _vq_kernel(x_ref, cbs_ref, gat_ref,
               out_ref, idx_ref, closs_ref, cn_ref):
    i = pl.program_id(0)

    @pl.when(i == 0)
    def _():
        for g in range(GROUPS):
            for q in range(NUM_Q):
                gat = gat_ref[g, q].astype(jnp.float32)
                cbf = (gat[:, :DG] + gat[:, DG:2 * DG]) + gat[:, 2 * DG:]
                col = _sumsq64(cbf)
                cn_ref[pl.ds(g * NUM_Q + q, 1), :] = col.T

    xb = x_ref[...]
    ids = lax.broadcasted_iota(jnp.int32, (TBLK, K), 1).astype(jnp.float32)

    residual = [xb[:, g * DG:(g + 1) * DG] for g in range(GROUPS)]
    qout = [jnp.zeros((TBLK, DG), jnp.float32) for _ in range(GROUPS)]
    closs_cols = [None] * (GROUPS * NUM_Q)
    for q in range(NUM_Q):
        for g in range(GROUPS):
            l = g * NUM_Q + q
            r = residual[g]
            rb2 = (r + r).astype(jnp.bfloat16)
            scores2 = lax.dot_general(
                rb2, cbs_ref[g, q], (((1,), (1,)), ((), ())),
                preferred_element_type=jnp.float32)
            rnorm = _sumsq64(r)
            d2 = (rnorm - scores2) + cn_ref[pl.ds(l, 1), :]
            m = jnp.min(d2, axis=1, keepdims=True)
            idxf = jnp.min(jnp.where(d2 <= m, ids, float(K)), axis=1,
                           keepdims=True)
            idx_ref[:, pl.ds(l, 1)] = idxf.astype(jnp.int32)
            onehot = (ids == idxf).astype(jnp.bfloat16)
            gq = lax.dot_general(
                onehot, gat_ref[g, q], (((1,), (0,)), ((), ())),
                preferred_element_type=jnp.float32)
            quant = (gq[:, :DG] + gq[:, DG:2 * DG]) + gq[:, 2 * DG:]
            new_r = r - quant
            closs_cols[l] = jnp.sum(new_r * new_r)
            qout[g] = qout[g] + quant
            residual[g] = new_r

    out_ref[...] = jnp.concatenate(qout, axis=1)
    closs_row = jnp.concatenate(
        [jnp.full((1, 1), c, jnp.float32) for c in closs_cols], axis=1)

    @pl.when(i == 0)
    def _():
        closs_ref[...] = closs_row

    @pl.when(i > 0)
    def _():
        closs_ref[...] = closs_ref[...] + closs_row


def _trunc_bf16(v):
    u = lax.bitcast_convert_type(v, jnp.uint32)
    return lax.bitcast_convert_type(u & jnp.uint32(0xFFFF0000), jnp.float32)


@jax.jit
def kernel(x, codebooks):
    B, N, D = x.shape
    x2 = x.reshape(TOKENS, D)
    cb_s = codebooks.astype(jnp.bfloat16)
    hi_f = _trunc_bf16(codebooks)
    r1 = codebooks - hi_f
    mid_f = _trunc_bf16(r1)
    lo_f = r1 - mid_f
    cb_gat = jnp.concatenate(
        [hi_f.astype(jnp.bfloat16), mid_f.astype(jnp.bfloat16),
         lo_f.astype(jnp.bfloat16)], axis=-1)
    grid = TOKENS // TBLK
    out, idx, closs = pl.pallas_call(
        _vq_kernel,
        grid=(grid,),
        in_specs=[
            pl.BlockSpec((TBLK, D), lambda i: (i, 0)),
            pl.BlockSpec((GROUPS, NUM_Q, K, DG), lambda i: (0, 0, 0, 0)),
            pl.BlockSpec((GROUPS, NUM_Q, K, 3 * DG), lambda i: (0, 0, 0, 0)),
        ],
        out_specs=[
            pl.BlockSpec((TBLK, D), lambda i: (i, 0)),
            pl.BlockSpec((TBLK, GROUPS * NUM_Q), lambda i: (i, 0)),
            pl.BlockSpec((1, GROUPS * NUM_Q), lambda i: (0, 0)),
        ],
        out_shape=[
            jax.ShapeDtypeStruct((TOKENS, D), jnp.float32),
            jax.ShapeDtypeStruct((TOKENS, GROUPS * NUM_Q), jnp.int32),
            jax.ShapeDtypeStruct((1, GROUPS * NUM_Q), jnp.float32),
        ],
        scratch_shapes=[pltpu.VMEM((GROUPS * NUM_Q, K), jnp.float32)],
    )(x2, cb_s, cb_gat)

    quantized = out.reshape(B, N, D)
    all_indices = idx.reshape(B, N, GROUPS, NUM_Q).transpose(2, 0, 1, 3)
    commit_losses = closs.reshape(GROUPS, NUM_Q) / (TOKENS * DG)
    return quantized, all_indices, commit_losses
